# baseline (speedup 1.0000x reference)
import functools

import jax
import jax.numpy as jnp
from jax import lax
from jax.experimental import pallas as pl
from jax.experimental.pallas import tpu as pltpu

N_ROWS = 16384
N_COLS = 1000
BLOCK_ROWS = 256
SUB = 128
N_SUB = BLOCK_ROWS // SUB
GRID = N_ROWS // BLOCK_ROWS
K = max(1, N_ROWS * 10 // 100)


def _fused_kernel(x_ref, tt_ref, out_ref, ce_s):
    b = pl.program_id(0)
    cols = []
    for g in range(N_SUB):
        xg = x_ref[g * SUB:(g + 1) * SUB, :]
        tg = tt_ref[0, :, g:g + 1]
        m = jnp.max(xg, axis=1, keepdims=True)
        s = jnp.sum(jnp.exp(xg - m), axis=1, keepdims=True)
        col = lax.broadcasted_iota(jnp.int32, xg.shape, 1)
        picked = jnp.sum(jnp.where(col == tg, xg, 0.0), axis=1, keepdims=True)
        cols.append((m - picked) + jnp.log(s))
    ce4 = jnp.concatenate(cols, axis=1)
    for bb in range(GRID):
        @pl.when(b == bb)
        def _store(bb=bb, ce4=ce4):
            ce_s[:, bb * N_SUB:(bb + 1) * N_SUB] = ce4

    @pl.when(b == GRID - 1)
    def _select():
        cev = ce_s[...]
        bits = lax.bitcast_convert_type(cev, jnp.int32)
        kf = jnp.float32(K)

        def body(_, carry):
            lo, hi = carry
            mid = lo + (hi - lo) // 2
            cnt = jnp.sum((bits >= mid).astype(jnp.int32))
            ge = cnt >= K
            return jnp.where(ge, mid, lo), jnp.where(ge, hi, mid)

        lo0 = jnp.int32(-1)
        hi0 = jnp.int32(0x7F800001)
        lo, _ = lax.fori_loop(0, 31, body, (lo0, hi0))

        gt = bits > lo
        cnt_gt = jnp.sum(gt.astype(jnp.float32))
        sum_gt = jnp.sum(jnp.where(gt, cev, 0.0))
        kth = jnp.max(jnp.where(bits == lo, cev, 0.0))
        out_ref[0, 0] = (sum_gt + kth * (kf - cnt_gt)) / kf


@functools.partial(jax.jit)
def kernel(inputs, targets):
    tt = targets.astype(jnp.int32).reshape(GRID, N_SUB, SUB).transpose(0, 2, 1)
    out = pl.pallas_call(
        _fused_kernel,
        grid=(GRID,),
        in_specs=[
            pl.BlockSpec((BLOCK_ROWS, N_COLS), lambda b: (b, 0)),
            pl.BlockSpec((1, SUB, N_SUB), lambda b: (b, 0, 0)),
        ],
        out_specs=pl.BlockSpec(memory_space=pltpu.SMEM),
        out_shape=jax.ShapeDtypeStruct((1, 1), jnp.float32),
        scratch_shapes=[pltpu.VMEM((SUB, SUB), jnp.float32)],
    )(inputs, tt)
    return out.reshape(())


# speedup vs baseline: 1.4117x; 1.4117x over previous
import functools

import jax
import jax.numpy as jnp
from jax import lax
from jax.experimental import pallas as pl
from jax.experimental.pallas import tpu as pltpu

N_ROWS = 16384
N_COLS = 1000
BLOCK_ROWS = 1024
GRID = N_ROWS // BLOCK_ROWS
K = max(1, N_ROWS * 10 // 100)


def _fused_kernel(x_ref, t_ref, out_ref, ce_s):
    b = pl.program_id(0)
    x = x_ref[...]
    t = t_ref[0]
    tcol = t.reshape(BLOCK_ROWS, 1)
    m = jnp.max(x, axis=1, keepdims=True)
    s = jnp.sum(jnp.exp(x - m), axis=1, keepdims=True)
    col = lax.broadcasted_iota(jnp.int32, x.shape, 1)
    picked = jnp.sum(jnp.where(col == tcol, x, 0.0), axis=1, keepdims=True)
    ce = (m - picked) + jnp.log(s)
    ce_s[pl.ds(b, 1), :] = ce.reshape(1, BLOCK_ROWS)

    @pl.when(b == GRID - 1)
    def _select():
        cev = ce_s[...]
        bits = lax.bitcast_convert_type(cev, jnp.int32)
        kf = jnp.float32(K)

        def body(_, carry):
            lo, hi = carry
            mid = lo + (hi - lo) // 2
            cnt = jnp.sum((bits >= mid).astype(jnp.int32))
            ge = cnt >= K
            return jnp.where(ge, mid, lo), jnp.where(ge, hi, mid)

        lo0 = jnp.int32(-1)
        hi0 = jnp.int32(0x7F800001)
        lo, _ = lax.fori_loop(0, 31, body, (lo0, hi0))

        gt = bits > lo
        cnt_gt = jnp.sum(gt.astype(jnp.float32))
        sum_gt = jnp.sum(jnp.where(gt, cev, 0.0))
        kth = jnp.max(jnp.where(bits == lo, cev, 0.0))
        out_ref[0, 0] = (sum_gt + kth * (kf - cnt_gt)) / kf


@functools.partial(jax.jit)
def kernel(inputs, targets):
    t3d = targets.astype(jnp.int32).reshape(GRID, 1, BLOCK_ROWS)
    out = pl.pallas_call(
        _fused_kernel,
        grid=(GRID,),
        in_specs=[
            pl.BlockSpec((BLOCK_ROWS, N_COLS), lambda b: (b, 0)),
            pl.BlockSpec((1, 1, BLOCK_ROWS), lambda b: (b, 0, 0)),
        ],
        out_specs=pl.BlockSpec(memory_space=pltpu.SMEM),
        out_shape=jax.ShapeDtypeStruct((1, 1), jnp.float32),
        scratch_shapes=[pltpu.VMEM((GRID, BLOCK_ROWS), jnp.float32)],
    )(inputs, t3d)
    return out.reshape(())


# speedup vs baseline: 1.4336x; 1.0156x over previous
import functools

import jax
import jax.numpy as jnp
from jax import lax
from jax.experimental import pallas as pl
from jax.experimental.pallas import tpu as pltpu

N_ROWS = 16384
N_COLS = 1000
BLOCK_ROWS = 2048
GRID = N_ROWS // BLOCK_ROWS
K = max(1, N_ROWS * 10 // 100)


def _fused_kernel(x_ref, t_ref, out_ref, ce_s):
    b = pl.program_id(0)
    x = x_ref[...]
    t = t_ref[0]
    tcol = t.reshape(BLOCK_ROWS, 1)
    m = jnp.max(x, axis=1, keepdims=True)
    s = jnp.sum(jnp.exp(x - m), axis=1, keepdims=True)
    col = lax.broadcasted_iota(jnp.int32, x.shape, 1)
    picked = jnp.sum(jnp.where(col == tcol, x, 0.0), axis=1, keepdims=True)
    ce = (m - picked) + jnp.log(s)
    ce_s[pl.ds(b, 1), :] = ce.reshape(1, BLOCK_ROWS)

    @pl.when(b == GRID - 1)
    def _select():
        cev = ce_s[...]
        bits = lax.bitcast_convert_type(cev, jnp.int32)
        kf = jnp.float32(K)

        def body(_, carry):
            lo, hi = carry
            mid = lo + (hi - lo) // 2
            cnt = jnp.sum((bits >= mid).astype(jnp.int32))
            ge = cnt >= K
            return jnp.where(ge, mid, lo), jnp.where(ge, hi, mid)

        lo0 = jnp.int32(-1)
        hi0 = jnp.int32(0x7F800001)
        lo, _ = lax.fori_loop(0, 31, body, (lo0, hi0))

        gt = bits > lo
        cnt_gt = jnp.sum(gt.astype(jnp.float32))
        sum_gt = jnp.sum(jnp.where(gt, cev, 0.0))
        kth = jnp.max(jnp.where(bits == lo, cev, 0.0))
        out_ref[0, 0] = (sum_gt + kth * (kf - cnt_gt)) / kf


@functools.partial(jax.jit)
def kernel(inputs, targets):
    t3d = targets.astype(jnp.int32).reshape(GRID, 1, BLOCK_ROWS)
    out = pl.pallas_call(
        _fused_kernel,
        grid=(GRID,),
        in_specs=[
            pl.BlockSpec((BLOCK_ROWS, N_COLS), lambda b: (b, 0)),
            pl.BlockSpec((1, 1, BLOCK_ROWS), lambda b: (b, 0, 0)),
        ],
        out_specs=pl.BlockSpec(memory_space=pltpu.SMEM),
        out_shape=jax.ShapeDtypeStruct((1, 1), jnp.float32),
        scratch_shapes=[pltpu.VMEM((GRID, BLOCK_ROWS), jnp.float32)],
    )(inputs, t3d)
    return out.reshape(())


# speedup vs baseline: 1.4546x; 1.0146x over previous
import functools

import jax
import jax.numpy as jnp
from jax import lax
from jax.experimental import pallas as pl
from jax.experimental.pallas import tpu as pltpu

N_ROWS = 16384
N_COLS = 1000
BLOCK_ROWS = 4096
GRID = N_ROWS // BLOCK_ROWS
K = max(1, N_ROWS * 10 // 100)


def _fused_kernel(x_ref, t_ref, out_ref, ce_s):
    b = pl.program_id(0)
    x = x_ref[...]
    t = t_ref[0]
    tcol = t.reshape(BLOCK_ROWS, 1)
    m = jnp.max(x, axis=1, keepdims=True)
    s = jnp.sum(jnp.exp(x - m), axis=1, keepdims=True)
    col = lax.broadcasted_iota(jnp.int32, x.shape, 1)
    picked = jnp.sum(jnp.where(col == tcol, x, 0.0), axis=1, keepdims=True)
    ce = (m - picked) + jnp.log(s)
    ce_s[pl.ds(b, 1), :] = ce.reshape(1, BLOCK_ROWS)

    @pl.when(b == GRID - 1)
    def _select():
        cev = ce_s[...]
        bits = lax.bitcast_convert_type(cev, jnp.int32)
        kf = jnp.float32(K)

        def body(_, carry):
            lo, hi = carry
            mid = lo + (hi - lo) // 2
            cnt = jnp.sum((bits >= mid).astype(jnp.int32))
            ge = cnt >= K
            return jnp.where(ge, mid, lo), jnp.where(ge, hi, mid)

        lo0 = jnp.int32(-1)
        hi0 = jnp.int32(0x7F800001)
        lo, _ = lax.fori_loop(0, 31, body, (lo0, hi0))

        gt = bits > lo
        cnt_gt = jnp.sum(gt.astype(jnp.float32))
        sum_gt = jnp.sum(jnp.where(gt, cev, 0.0))
        kth = jnp.max(jnp.where(bits == lo, cev, 0.0))
        out_ref[0, 0] = (sum_gt + kth * (kf - cnt_gt)) / kf


@functools.partial(jax.jit)
def kernel(inputs, targets):
    t3d = targets.astype(jnp.int32).reshape(GRID, 1, BLOCK_ROWS)
    out = pl.pallas_call(
        _fused_kernel,
        grid=(GRID,),
        in_specs=[
            pl.BlockSpec((BLOCK_ROWS, N_COLS), lambda b: (b, 0)),
            pl.BlockSpec((1, 1, BLOCK_ROWS), lambda b: (b, 0, 0)),
        ],
        out_specs=pl.BlockSpec(memory_space=pltpu.SMEM),
        out_shape=jax.ShapeDtypeStruct((1, 1), jnp.float32),
        scratch_shapes=[pltpu.VMEM((GRID, BLOCK_ROWS), jnp.float32)],
    )(inputs, t3d)
    return out.reshape(())
